# baseline (speedup 1.0000x reference)
import functools

import jax
import jax.numpy as jnp
from jax import lax
from jax.experimental import pallas as pl
from jax.experimental.pallas import tpu as pltpu
from jax.experimental.pallas import tpu_sc as plsc

N = 10000
E = 320000
D_IN = 128
D_HID = 256
D_OUT = 128

NP_ = 10240
EP = 323584
DUMMY = N
CHUNK = 64
RPS = NP_ // 16
BR = 512


_SC_PARAMS = pltpu.CompilerParams(needs_layout_passes=False)


def _deg_body(src_hbm, dst_hbm, out_s_hbm, out_d_hbm,
              ebuf_s, ebuf_d, cnt_s, cnt_d):
    c = lax.axis_index("c")
    s = lax.axis_index("s")
    wid = c * 16 + s
    zeros16 = jnp.zeros((16,), jnp.float32)
    ones16 = jnp.full((16,), 1.0, jnp.float32)

    def zrow(r, _):
        cnt_s[pl.ds(r * 16, 16)] = zeros16
        cnt_d[pl.ds(r * 16, 16)] = zeros16
        return 0
    lax.fori_loop(0, NP_ // 16, zrow, 0)

    for k in range(5):
        base = wid * 10000 + k * 2000
        pltpu.sync_copy(src_hbm.at[pl.ds(base, 2000)], ebuf_s)
        pltpu.sync_copy(dst_hbm.at[pl.ds(base, 2000)], ebuf_d)

        def cbody(i, _):
            sid = ebuf_s[pl.ds(i * 16, 16)]
            did = ebuf_d[pl.ds(i * 16, 16)]
            plsc.addupdate_scatter(cnt_s, [sid], ones16)
            plsc.addupdate_scatter(cnt_d, [did], ones16)
            return 0
        lax.fori_loop(0, 125, cbody, 0)

    pltpu.sync_copy(cnt_s, out_s_hbm.at[pl.ds(wid * NP_, NP_)])
    pltpu.sync_copy(cnt_d, out_d_hbm.at[pl.ds(wid * NP_, NP_)])


def _degrees(src, dst):
    mesh = plsc.VectorSubcoreMesh(core_axis_name="c", subcore_axis_name="s")
    f = pl.kernel(
        _deg_body,
        out_type=[jax.ShapeDtypeStruct((32 * NP_,), jnp.float32),
                  jax.ShapeDtypeStruct((32 * NP_,), jnp.float32)],
        mesh=mesh,
        compiler_params=_SC_PARAMS,
        scratch_types=[
            pltpu.VMEM((2000,), jnp.int32),
            pltpu.VMEM((2000,), jnp.int32),
            pltpu.VMEM((NP_,), jnp.float32),
            pltpu.VMEM((NP_,), jnp.float32),
        ],
    )
    cs, cd = f(src, dst)
    return cs.reshape(32, NP_), cd.reshape(32, NP_)


def _zero_acc(rows, acc, s):
    zeros16 = jnp.zeros((16,), jnp.float32)

    def zrow(r, _):
        for j in range(8):
            rows[r, pl.ds(16 * j, 16)] = zeros16
        return 0
    lax.fori_loop(0, CHUNK, zrow, 0)
    for k in range(RPS // CHUNK):
        pltpu.sync_copy(rows, acc.at[pl.ds(s * RPS + k * CHUNK, CHUNK)])


NB = 4


def _agg_chunks(h_hbm, srcp, dstp, bufs, acc, tile_base, n_chunks):
    def one(j, buf):
        si, di, ro, se = buf
        base = tile_base + j * CHUNK
        pltpu.sync_copy(srcp.at[pl.ds(base, CHUNK)], si)
        pltpu.sync_copy(dstp.at[pl.ds(base, CHUNK)], di)
        return pltpu.async_copy(h_hbm.at[si], ro, se)

    def scat(buf):
        si, di, ro, se = buf
        pltpu.sync_copy(ro, acc.at[di], add=True)

    def group(cbase, count):
        gd = [one(cbase + 0, bufs[0])]
        if count > 1:
            gd.append(one(cbase + 1, bufs[1]))
        for i in range(count):
            gd[i].wait()
            if i + 2 < count:
                gd.append(one(cbase + i + 2, bufs[i + 2]))
            scat(bufs[i])

    def body(q, _):
        group(q * NB, NB)
        return 0
    lax.fori_loop(0, n_chunks // NB, body, 0)
    if n_chunks % NB:
        group(n_chunks - n_chunks % NB, n_chunks % NB)


def _agg_edges_body(h_hbm, srcp, dstp, out_hbm, *scr):
    c = lax.axis_index("c")
    s = lax.axis_index("s")
    bufs = [tuple(scr[4 * i:4 * i + 4]) for i in range(NB)]
    acc = scr[4 * NB]
    _zero_acc(bufs[0][2], acc, s)
    plsc.subcore_barrier()
    n_chunks = EP // 32 // CHUNK
    tile_base = (c * 16 + s) * (EP // 32)
    _agg_chunks(h_hbm, srcp, dstp, bufs, acc, tile_base, n_chunks)
    plsc.subcore_barrier()
    pltpu.sync_copy(acc.at[pl.ds(s * RPS, RPS)],
                    out_hbm.at[c, pl.ds(s * RPS, RPS)])


def _agg_cols_body(hlo_hbm, hhi_hbm, srcp, dstp, outlo_hbm, outhi_hbm,
                   *scr):
    c = lax.axis_index("c")
    s = lax.axis_index("s")
    bufs = [tuple(scr[4 * i:4 * i + 4]) for i in range(NB)]
    acc = scr[4 * NB]
    _zero_acc(bufs[0][2], acc, s)
    plsc.subcore_barrier()
    n_chunks = EP // 16 // CHUNK
    tile_base = s * (EP // 16)

    @pl.when(c == 0)
    def _():
        _agg_chunks(hlo_hbm, srcp, dstp, bufs, acc, tile_base, n_chunks)

    @pl.when(c == 1)
    def _():
        _agg_chunks(hhi_hbm, srcp, dstp, bufs, acc, tile_base, n_chunks)

    plsc.subcore_barrier()

    @pl.when(c == 0)
    def _():
        pltpu.sync_copy(acc.at[pl.ds(s * RPS, RPS)],
                        outlo_hbm.at[pl.ds(s * RPS, RPS)])

    @pl.when(c == 1)
    def _():
        pltpu.sync_copy(acc.at[pl.ds(s * RPS, RPS)],
                        outhi_hbm.at[pl.ds(s * RPS, RPS)])


def _agg_scratch():
    scr = []
    for _ in range(NB):
        scr += [pltpu.VMEM((CHUNK,), jnp.int32),
                pltpu.VMEM((CHUNK,), jnp.int32),
                pltpu.VMEM((CHUNK, 128), jnp.float32),
                pltpu.SemaphoreType.DMA]
    scr.append(pltpu.VMEM_SHARED((NP_, 128), jnp.float32))
    return scr


def _agg_edges(h, srcp, dstp):
    mesh = plsc.VectorSubcoreMesh(core_axis_name="c", subcore_axis_name="s")
    f = pl.kernel(
        _agg_edges_body,
        out_type=jax.ShapeDtypeStruct((2, NP_, 128), jnp.float32),
        mesh=mesh,
        compiler_params=_SC_PARAMS,
        scratch_types=_agg_scratch(),
    )
    return f(h, srcp, dstp)


def _agg_cols(hlo, hhi, srcp, dstp):
    mesh = plsc.VectorSubcoreMesh(core_axis_name="c", subcore_axis_name="s")
    f = pl.kernel(
        _agg_cols_body,
        out_type=[jax.ShapeDtypeStruct((NP_, 128), jnp.float32),
                  jax.ShapeDtypeStruct((NP_, 128), jnp.float32)],
        mesh=mesh,
        compiler_params=_SC_PARAMS,
        scratch_types=_agg_scratch(),
    )
    return f(hlo, hhi, srcp, dstp)


def _norm(cnt_blk):
    return lax.rsqrt(jnp.sum(cnt_blk[...], axis=0) + 1.0)


def _tc_a_body(feat, cs, h1):
    on = _norm(cs)
    h1[...] = feat[...] * on[:, None]


def _tc_b_body(p1, h1, cs, cd, w1, b1, h2lo, h2hi):
    on, inn = _norm(cs), _norm(cd)
    aggf = p1[0] + p1[1] + h1[...]
    x1 = jnp.dot(aggf * inn[:, None], w1[...],
                 preferred_element_type=jnp.float32) + b1[...]
    h2 = jnp.maximum(x1, 0.0) * on[:, None]
    h2lo[...] = h2[:, :128]
    h2hi[...] = h2[:, 128:]


def _tc_c_body(a2lo, a2hi, h2lo, h2hi, cs, cd, w2, b2, w3, g3):
    on, inn = _norm(cs), _norm(cd)
    aggf = jnp.concatenate([a2lo[...] + h2lo[...], a2hi[...] + h2hi[...]],
                           axis=1)
    x2 = jnp.maximum(
        jnp.dot(aggf * inn[:, None], w2[...],
                preferred_element_type=jnp.float32) + b2[...], 0.0)
    g3[...] = jnp.dot(x2 * on[:, None], w3[...],
                      preferred_element_type=jnp.float32)


def _tc_d_body(p3, g3, cd, b3, out):
    inn = _norm(cd)
    aggf = p3[0] + p3[1] + g3[...]
    out[...] = aggf * inn[:, None] + b3[...]


def _rows_spec(d=128):
    return pl.BlockSpec((BR, d), lambda i: (i, 0))


def _plane_spec():
    return pl.BlockSpec((2, BR, 128), lambda i: (0, i, 0))


def _cnt_spec():
    return pl.BlockSpec((32, BR), lambda i: (0, i))


def _full_spec(shape):
    return pl.BlockSpec(shape, lambda i: tuple(0 for _ in shape))


def _tc_call(body, in_specs, out_specs, out_shapes, args):
    return pl.pallas_call(
        body,
        grid=(NP_ // BR,),
        in_specs=in_specs,
        out_specs=out_specs,
        out_shape=out_shapes,
    )(*args)


def kernel(features, edge_index, W1, b1, W2, b2, W3, b3):
    src = edge_index[0].astype(jnp.int32)
    dst = edge_index[1].astype(jnp.int32)
    pad = jnp.full((EP - E,), DUMMY, jnp.int32)
    srcp = jnp.concatenate([src, pad])
    dstp = jnp.concatenate([dst, pad])
    feat_p = jnp.pad(features, ((0, NP_ - N), (0, 0)))
    b1r = b1.reshape(1, -1)
    b2r = b2.reshape(1, -1)
    b3r = b3.reshape(1, -1)

    cs, cd = _degrees(src, dst)

    h1 = _tc_call(
        _tc_a_body,
        [_rows_spec(), _cnt_spec()],
        _rows_spec(),
        jax.ShapeDtypeStruct((NP_, 128), jnp.float32),
        (feat_p, cs),
    )

    p1 = _agg_edges(h1, srcp, dstp)

    h2lo, h2hi = _tc_call(
        _tc_b_body,
        [_plane_spec(), _rows_spec(), _cnt_spec(), _cnt_spec(),
         _full_spec((D_IN, D_HID)), _full_spec((1, D_HID))],
        [_rows_spec(), _rows_spec()],
        [jax.ShapeDtypeStruct((NP_, 128), jnp.float32),
         jax.ShapeDtypeStruct((NP_, 128), jnp.float32)],
        (p1, h1, cs, cd, W1, b1r),
    )

    a2lo, a2hi = _agg_cols(h2lo, h2hi, srcp, dstp)

    g3 = _tc_call(
        _tc_c_body,
        [_rows_spec(), _rows_spec(), _rows_spec(), _rows_spec(),
         _cnt_spec(), _cnt_spec(), _full_spec((D_HID, D_HID)),
         _full_spec((1, D_HID)), _full_spec((D_HID, D_OUT))],
        _rows_spec(),
        jax.ShapeDtypeStruct((NP_, 128), jnp.float32),
        (a2lo, a2hi, h2lo, h2hi, cs, cd, W2, b2r, W3),
    )

    p3 = _agg_edges(g3, srcp, dstp)

    out = _tc_call(
        _tc_d_body,
        [_plane_spec(), _rows_spec(), _cnt_spec(), _full_spec((1, D_OUT))],
        _rows_spec(),
        jax.ShapeDtypeStruct((NP_, 128), jnp.float32),
        (p3, g3, cd, b3r),
    )
    return out[:N]


# speedup vs baseline: 1.0989x; 1.0989x over previous
import functools

import jax
import jax.numpy as jnp
from jax import lax
from jax.experimental import pallas as pl
from jax.experimental.pallas import tpu as pltpu
from jax.experimental.pallas import tpu_sc as plsc

N = 10000
E = 320000
D_IN = 128
D_HID = 256
D_OUT = 128

NP_ = 10240
EP = 323584
DUMMY = N
CHUNK = 128
RPS = NP_ // 16
BR = 512


_SC_PARAMS = pltpu.CompilerParams(needs_layout_passes=False)


def _deg_body(src_hbm, dst_hbm, out_s_hbm, out_d_hbm,
              ebuf_s, ebuf_d, cnt_s, cnt_d):
    c = lax.axis_index("c")
    s = lax.axis_index("s")
    wid = c * 16 + s
    zeros16 = jnp.zeros((16,), jnp.float32)
    ones16 = jnp.full((16,), 1.0, jnp.float32)

    def zrow(r, _):
        cnt_s[pl.ds(r * 16, 16)] = zeros16
        cnt_d[pl.ds(r * 16, 16)] = zeros16
        return 0
    lax.fori_loop(0, NP_ // 16, zrow, 0)

    for k in range(5):
        base = wid * 10000 + k * 2000
        pltpu.sync_copy(src_hbm.at[pl.ds(base, 2000)], ebuf_s)
        pltpu.sync_copy(dst_hbm.at[pl.ds(base, 2000)], ebuf_d)

        def cbody(i, _):
            sid = ebuf_s[pl.ds(i * 16, 16)]
            did = ebuf_d[pl.ds(i * 16, 16)]
            plsc.addupdate_scatter(cnt_s, [sid], ones16)
            plsc.addupdate_scatter(cnt_d, [did], ones16)
            return 0
        lax.fori_loop(0, 125, cbody, 0)

    pltpu.sync_copy(cnt_s, out_s_hbm.at[pl.ds(wid * NP_, NP_)])
    pltpu.sync_copy(cnt_d, out_d_hbm.at[pl.ds(wid * NP_, NP_)])


def _degrees(src, dst):
    mesh = plsc.VectorSubcoreMesh(core_axis_name="c", subcore_axis_name="s")
    f = pl.kernel(
        _deg_body,
        out_type=[jax.ShapeDtypeStruct((32 * NP_,), jnp.float32),
                  jax.ShapeDtypeStruct((32 * NP_,), jnp.float32)],
        mesh=mesh,
        compiler_params=_SC_PARAMS,
        scratch_types=[
            pltpu.VMEM((2000,), jnp.int32),
            pltpu.VMEM((2000,), jnp.int32),
            pltpu.VMEM((NP_,), jnp.float32),
            pltpu.VMEM((NP_,), jnp.float32),
        ],
    )
    cs, cd = f(src, dst)
    return cs.reshape(32, NP_), cd.reshape(32, NP_)


def _zero_acc(rows, acc, s):
    zeros16 = jnp.zeros((16,), jnp.float32)

    def zrow(r, _):
        for j in range(8):
            rows[r, pl.ds(16 * j, 16)] = zeros16
        return 0
    lax.fori_loop(0, CHUNK, zrow, 0)
    for k in range(RPS // CHUNK):
        pltpu.sync_copy(rows, acc.at[pl.ds(s * RPS + k * CHUNK, CHUNK)])


def _agg_chunks(h_hbm, srcp, dstp, sidx, didx, rows, acc, sem, tile_base,
                n_chunks, sidx2, didx2, rows2, sem2):
    def one(j, si, di, ro, se):
        base = tile_base + j * CHUNK
        pltpu.sync_copy(srcp.at[pl.ds(base, CHUNK)], si)
        pltpu.sync_copy(dstp.at[pl.ds(base, CHUNK)], di)
        return pltpu.async_copy(h_hbm.at[si], ro, se)

    def body(q, _):
        ga = one(2 * q, sidx, didx, rows, sem)
        gb = one(2 * q + 1, sidx2, didx2, rows2, sem2)
        ga.wait()
        pltpu.sync_copy(rows, acc.at[didx], add=True)
        gb.wait()
        pltpu.sync_copy(rows2, acc.at[didx2], add=True)
        return 0
    lax.fori_loop(0, n_chunks // 2, body, 0)
    if n_chunks % 2:
        one(n_chunks - 1, sidx, didx, rows, sem).wait()
        pltpu.sync_copy(rows, acc.at[didx], add=True)


def _agg_edges_body(h_hbm, srcp, dstp, out_hbm, sidx, didx, rows, acc, sem,
                    sidx2, didx2, rows2, sem2):
    c = lax.axis_index("c")
    s = lax.axis_index("s")
    _zero_acc(rows, acc, s)
    plsc.subcore_barrier()
    n_chunks = EP // 32 // CHUNK
    tile_base = (c * 16 + s) * (EP // 32)
    _agg_chunks(h_hbm, srcp, dstp, sidx, didx, rows, acc, sem, tile_base,
                n_chunks, sidx2, didx2, rows2, sem2)
    plsc.subcore_barrier()
    pltpu.sync_copy(acc.at[pl.ds(s * RPS, RPS)],
                    out_hbm.at[c, pl.ds(s * RPS, RPS)])


def _agg_cols_body(hlo_hbm, hhi_hbm, srcp, dstp, outlo_hbm, outhi_hbm,
                   sidx, didx, rows, acc, sem, sidx2, didx2, rows2, sem2):
    c = lax.axis_index("c")
    s = lax.axis_index("s")
    _zero_acc(rows, acc, s)
    plsc.subcore_barrier()
    n_chunks = EP // 16 // CHUNK
    tile_base = s * (EP // 16)

    @pl.when(c == 0)
    def _():
        _agg_chunks(hlo_hbm, srcp, dstp, sidx, didx, rows, acc, sem,
                    tile_base, n_chunks, sidx2, didx2, rows2, sem2)

    @pl.when(c == 1)
    def _():
        _agg_chunks(hhi_hbm, srcp, dstp, sidx, didx, rows, acc, sem,
                    tile_base, n_chunks, sidx2, didx2, rows2, sem2)

    plsc.subcore_barrier()

    @pl.when(c == 0)
    def _():
        pltpu.sync_copy(acc.at[pl.ds(s * RPS, RPS)],
                        outlo_hbm.at[pl.ds(s * RPS, RPS)])

    @pl.when(c == 1)
    def _():
        pltpu.sync_copy(acc.at[pl.ds(s * RPS, RPS)],
                        outhi_hbm.at[pl.ds(s * RPS, RPS)])


def _agg_scratch():
    return [
        pltpu.VMEM((CHUNK,), jnp.int32),
        pltpu.VMEM((CHUNK,), jnp.int32),
        pltpu.VMEM((CHUNK, 128), jnp.float32),
        pltpu.VMEM_SHARED((NP_, 128), jnp.float32),
        pltpu.SemaphoreType.DMA,
        pltpu.VMEM((CHUNK,), jnp.int32),
        pltpu.VMEM((CHUNK,), jnp.int32),
        pltpu.VMEM((CHUNK, 128), jnp.float32),
        pltpu.SemaphoreType.DMA,
    ]


def _agg_edges(h, srcp, dstp):
    mesh = plsc.VectorSubcoreMesh(core_axis_name="c", subcore_axis_name="s")
    f = pl.kernel(
        _agg_edges_body,
        out_type=jax.ShapeDtypeStruct((2, NP_, 128), jnp.float32),
        mesh=mesh,
        compiler_params=_SC_PARAMS,
        scratch_types=_agg_scratch(),
    )
    return f(h, srcp, dstp)


def _agg_cols(hlo, hhi, srcp, dstp):
    mesh = plsc.VectorSubcoreMesh(core_axis_name="c", subcore_axis_name="s")
    f = pl.kernel(
        _agg_cols_body,
        out_type=[jax.ShapeDtypeStruct((NP_, 128), jnp.float32),
                  jax.ShapeDtypeStruct((NP_, 128), jnp.float32)],
        mesh=mesh,
        compiler_params=_SC_PARAMS,
        scratch_types=_agg_scratch(),
    )
    return f(hlo, hhi, srcp, dstp)


def _norm(cnt_blk):
    return lax.rsqrt(jnp.sum(cnt_blk[...], axis=0) + 1.0)


def _tc_a_body(feat, cs, h1):
    on = _norm(cs)
    h1[...] = feat[...] * on[:, None]


def _tc_b_body(p1, h1, cs, cd, w1, b1, h2lo, h2hi):
    on, inn = _norm(cs), _norm(cd)
    aggf = p1[0] + p1[1] + h1[...]
    x1 = jnp.dot(aggf * inn[:, None], w1[...],
                 preferred_element_type=jnp.float32) + b1[...]
    h2 = jnp.maximum(x1, 0.0) * on[:, None]
    h2lo[...] = h2[:, :128]
    h2hi[...] = h2[:, 128:]


def _tc_c_body(a2lo, a2hi, h2lo, h2hi, cs, cd, w2, b2, w3, g3):
    on, inn = _norm(cs), _norm(cd)
    aggf = jnp.concatenate([a2lo[...] + h2lo[...], a2hi[...] + h2hi[...]],
                           axis=1)
    x2 = jnp.maximum(
        jnp.dot(aggf * inn[:, None], w2[...],
                preferred_element_type=jnp.float32) + b2[...], 0.0)
    g3[...] = jnp.dot(x2 * on[:, None], w3[...],
                      preferred_element_type=jnp.float32)


def _tc_d_body(p3, g3, cd, b3, out):
    inn = _norm(cd)
    aggf = p3[0] + p3[1] + g3[...]
    out[...] = aggf * inn[:, None] + b3[...]


def _rows_spec(d=128):
    return pl.BlockSpec((BR, d), lambda i: (i, 0))


def _plane_spec():
    return pl.BlockSpec((2, BR, 128), lambda i: (0, i, 0))


def _cnt_spec():
    return pl.BlockSpec((32, BR), lambda i: (0, i))


def _full_spec(shape):
    return pl.BlockSpec(shape, lambda i: tuple(0 for _ in shape))


def _tc_call(body, in_specs, out_specs, out_shapes, args):
    return pl.pallas_call(
        body,
        grid=(NP_ // BR,),
        in_specs=in_specs,
        out_specs=out_specs,
        out_shape=out_shapes,
    )(*args)


def kernel(features, edge_index, W1, b1, W2, b2, W3, b3):
    src = edge_index[0].astype(jnp.int32)
    dst = edge_index[1].astype(jnp.int32)
    pad = jnp.full((EP - E,), DUMMY, jnp.int32)
    srcp = jnp.concatenate([src, pad])
    dstp = jnp.concatenate([dst, pad])
    feat_p = jnp.pad(features, ((0, NP_ - N), (0, 0)))
    b1r = b1.reshape(1, -1)
    b2r = b2.reshape(1, -1)
    b3r = b3.reshape(1, -1)

    cs, cd = _degrees(src, dst)

    h1 = _tc_call(
        _tc_a_body,
        [_rows_spec(), _cnt_spec()],
        _rows_spec(),
        jax.ShapeDtypeStruct((NP_, 128), jnp.float32),
        (feat_p, cs),
    )

    p1 = _agg_edges(h1, srcp, dstp)

    h2lo, h2hi = _tc_call(
        _tc_b_body,
        [_plane_spec(), _rows_spec(), _cnt_spec(), _cnt_spec(),
         _full_spec((D_IN, D_HID)), _full_spec((1, D_HID))],
        [_rows_spec(), _rows_spec()],
        [jax.ShapeDtypeStruct((NP_, 128), jnp.float32),
         jax.ShapeDtypeStruct((NP_, 128), jnp.float32)],
        (p1, h1, cs, cd, W1, b1r),
    )

    a2lo, a2hi = _agg_cols(h2lo, h2hi, srcp, dstp)

    g3 = _tc_call(
        _tc_c_body,
        [_rows_spec(), _rows_spec(), _rows_spec(), _rows_spec(),
         _cnt_spec(), _cnt_spec(), _full_spec((D_HID, D_HID)),
         _full_spec((1, D_HID)), _full_spec((D_HID, D_OUT))],
        _rows_spec(),
        jax.ShapeDtypeStruct((NP_, 128), jnp.float32),
        (a2lo, a2hi, h2lo, h2hi, cs, cd, W2, b2r, W3),
    )

    p3 = _agg_edges(g3, srcp, dstp)

    out = _tc_call(
        _tc_d_body,
        [_plane_spec(), _rows_spec(), _cnt_spec(), _full_spec((1, D_OUT))],
        _rows_spec(),
        jax.ShapeDtypeStruct((NP_, 128), jnp.float32),
        (p3, g3, cd, b3r),
    )
    return out[:N]


# speedup vs baseline: 1.1425x; 1.0397x over previous
import jax
import jax.numpy as jnp
from jax import lax
from jax.experimental import pallas as pl
from jax.experimental.pallas import tpu as pltpu
from jax.experimental.pallas import tpu_sc as plsc

N = 10000
E = 320000
D_IN = 128
D_HID = 256
D_OUT = 128

NP_ = 10240
EP = 323584
DUMMY = N
CHUNK = 128
RPS = NP_ // 16
BR = 512


_SC_PARAMS = pltpu.CompilerParams(needs_layout_passes=False)


def _deg_body(src_hbm, dst_hbm, out_s_hbm, out_d_hbm,
              ebuf_s, ebuf_d, cnt_s, cnt_d):
    c = lax.axis_index("c")
    s = lax.axis_index("s")
    wid = c * 16 + s
    zeros16 = jnp.zeros((16,), jnp.float32)
    ones16 = jnp.full((16,), 1.0, jnp.float32)

    def zrow(r, _):
        cnt_s[pl.ds(r * 16, 16)] = zeros16
        cnt_d[pl.ds(r * 16, 16)] = zeros16
        return 0
    lax.fori_loop(0, NP_ // 16, zrow, 0)

    for k in range(5):
        base = wid * 10000 + k * 2000
        pltpu.sync_copy(src_hbm.at[pl.ds(base, 2000)], ebuf_s)
        pltpu.sync_copy(dst_hbm.at[pl.ds(base, 2000)], ebuf_d)

        def cbody(i, _):
            sid = ebuf_s[pl.ds(i * 16, 16)]
            did = ebuf_d[pl.ds(i * 16, 16)]
            plsc.addupdate_scatter(cnt_s, [sid], ones16)
            plsc.addupdate_scatter(cnt_d, [did], ones16)
            return 0
        lax.fori_loop(0, 125, cbody, 0)

    pltpu.sync_copy(cnt_s, out_s_hbm.at[pl.ds(wid * NP_, NP_)])
    pltpu.sync_copy(cnt_d, out_d_hbm.at[pl.ds(wid * NP_, NP_)])


def _degrees(src, dst):
    mesh = plsc.VectorSubcoreMesh(core_axis_name="c", subcore_axis_name="s")
    f = pl.kernel(
        _deg_body,
        out_type=[jax.ShapeDtypeStruct((32 * NP_,), jnp.float32),
                  jax.ShapeDtypeStruct((32 * NP_,), jnp.float32)],
        mesh=mesh,
        compiler_params=_SC_PARAMS,
        scratch_types=[
            pltpu.VMEM((2000,), jnp.int32),
            pltpu.VMEM((2000,), jnp.int32),
            pltpu.VMEM((NP_,), jnp.float32),
            pltpu.VMEM((NP_,), jnp.float32),
        ],
    )
    cs, cd = f(src, dst)
    return cs.reshape(32, NP_), cd.reshape(32, NP_)


def _zero_acc(rows, acc, s):
    zeros16 = jnp.zeros((16,), jnp.float32)

    def zrow(r, _):
        for j in range(8):
            rows[r, pl.ds(16 * j, 16)] = zeros16
        return 0
    lax.fori_loop(0, CHUNK, zrow, 0)
    for k in range(RPS // CHUNK):
        pltpu.sync_copy(rows, acc.at[pl.ds(s * RPS + k * CHUNK, CHUNK)])


def _agg_chunks(h_hbm, srcp, dstp, sidx, didx, rows, acc, sem, tile_base,
                n_chunks, sidx2, didx2, rows2, sem2):
    def one(j, si, di, ro, se):
        base = tile_base + j * CHUNK
        pltpu.sync_copy(srcp.at[pl.ds(base, CHUNK)], si)
        pltpu.sync_copy(dstp.at[pl.ds(base, CHUNK)], di)
        return pltpu.async_copy(h_hbm.at[si], ro, se)

    def body(q, _):
        ga = one(4 * q, sidx, didx, rows, sem)
        gb = one(4 * q + 1, sidx2, didx2, rows2, sem2)
        ga.wait()
        pltpu.sync_copy(rows, acc.at[didx], add=True)
        ga2 = one(4 * q + 2, sidx, didx, rows, sem)
        gb.wait()
        pltpu.sync_copy(rows2, acc.at[didx2], add=True)
        gb2 = one(4 * q + 3, sidx2, didx2, rows2, sem2)
        ga2.wait()
        pltpu.sync_copy(rows, acc.at[didx], add=True)
        gb2.wait()
        pltpu.sync_copy(rows2, acc.at[didx2], add=True)
        return 0
    lax.fori_loop(0, n_chunks // 4, body, 0)
    rem_base = n_chunks - n_chunks % 4
    if n_chunks % 4 >= 2:
        ga = one(rem_base, sidx, didx, rows, sem)
        gb = one(rem_base + 1, sidx2, didx2, rows2, sem2)
        ga.wait()
        pltpu.sync_copy(rows, acc.at[didx], add=True)
        gb.wait()
        pltpu.sync_copy(rows2, acc.at[didx2], add=True)
    if n_chunks % 2:
        one(n_chunks - 1, sidx, didx, rows, sem).wait()
        pltpu.sync_copy(rows, acc.at[didx], add=True)


def _agg_edges_body(h_hbm, srcp, dstp, out_hbm, sidx, didx, rows, acc, sem,
                    sidx2, didx2, rows2, sem2):
    c = lax.axis_index("c")
    s = lax.axis_index("s")
    _zero_acc(rows, acc, s)
    plsc.subcore_barrier()
    n_chunks = EP // 32 // CHUNK
    tile_base = (c * 16 + s) * (EP // 32)
    _agg_chunks(h_hbm, srcp, dstp, sidx, didx, rows, acc, sem, tile_base,
                n_chunks, sidx2, didx2, rows2, sem2)
    plsc.subcore_barrier()
    pltpu.sync_copy(acc.at[pl.ds(s * RPS, RPS)],
                    out_hbm.at[c, pl.ds(s * RPS, RPS)])


def _agg_cols_body(hlo_hbm, hhi_hbm, srcp, dstp, outlo_hbm, outhi_hbm,
                   sidx, didx, rows, acc, sem, sidx2, didx2, rows2, sem2):
    c = lax.axis_index("c")
    s = lax.axis_index("s")
    _zero_acc(rows, acc, s)
    plsc.subcore_barrier()
    n_chunks = EP // 16 // CHUNK
    tile_base = s * (EP // 16)

    @pl.when(c == 0)
    def _():
        _agg_chunks(hlo_hbm, srcp, dstp, sidx, didx, rows, acc, sem,
                    tile_base, n_chunks, sidx2, didx2, rows2, sem2)

    @pl.when(c == 1)
    def _():
        _agg_chunks(hhi_hbm, srcp, dstp, sidx, didx, rows, acc, sem,
                    tile_base, n_chunks, sidx2, didx2, rows2, sem2)

    plsc.subcore_barrier()

    @pl.when(c == 0)
    def _():
        pltpu.sync_copy(acc.at[pl.ds(s * RPS, RPS)],
                        outlo_hbm.at[pl.ds(s * RPS, RPS)])

    @pl.when(c == 1)
    def _():
        pltpu.sync_copy(acc.at[pl.ds(s * RPS, RPS)],
                        outhi_hbm.at[pl.ds(s * RPS, RPS)])


def _agg_scratch():
    return [
        pltpu.VMEM((CHUNK,), jnp.int32),
        pltpu.VMEM((CHUNK,), jnp.int32),
        pltpu.VMEM((CHUNK, 128), jnp.float32),
        pltpu.VMEM_SHARED((NP_, 128), jnp.float32),
        pltpu.SemaphoreType.DMA,
        pltpu.VMEM((CHUNK,), jnp.int32),
        pltpu.VMEM((CHUNK,), jnp.int32),
        pltpu.VMEM((CHUNK, 128), jnp.float32),
        pltpu.SemaphoreType.DMA,
    ]


def _agg_edges(h, srcp, dstp):
    mesh = plsc.VectorSubcoreMesh(core_axis_name="c", subcore_axis_name="s")
    f = pl.kernel(
        _agg_edges_body,
        out_type=jax.ShapeDtypeStruct((2, NP_, 128), jnp.float32),
        mesh=mesh,
        compiler_params=_SC_PARAMS,
        scratch_types=_agg_scratch(),
    )
    return f(h, srcp, dstp)


def _agg_cols(hlo, hhi, srcp, dstp):
    mesh = plsc.VectorSubcoreMesh(core_axis_name="c", subcore_axis_name="s")
    f = pl.kernel(
        _agg_cols_body,
        out_type=[jax.ShapeDtypeStruct((NP_, 128), jnp.float32),
                  jax.ShapeDtypeStruct((NP_, 128), jnp.float32)],
        mesh=mesh,
        compiler_params=_SC_PARAMS,
        scratch_types=_agg_scratch(),
    )
    return f(hlo, hhi, srcp, dstp)


def _norm(cnt_blk):
    return lax.rsqrt(jnp.sum(cnt_blk[...], axis=0) + 1.0)


def _tc_a_body(feat, cs, h1):
    on = _norm(cs)
    h1[...] = feat[...] * on[:, None]


def _tc_b_body(p1, h1, cs, cd, w1, b1, h2lo, h2hi):
    on, inn = _norm(cs), _norm(cd)
    aggf = p1[0] + p1[1] + h1[...]
    x1 = jnp.dot(aggf * inn[:, None], w1[...],
                 preferred_element_type=jnp.float32) + b1[...]
    h2 = jnp.maximum(x1, 0.0) * on[:, None]
    h2lo[...] = h2[:, :128]
    h2hi[...] = h2[:, 128:]


def _tc_c_body(a2lo, a2hi, h2lo, h2hi, cs, cd, w2, b2, w3, g3):
    on, inn = _norm(cs), _norm(cd)
    aggf = jnp.concatenate([a2lo[...] + h2lo[...], a2hi[...] + h2hi[...]],
                           axis=1)
    x2 = jnp.maximum(
        jnp.dot(aggf * inn[:, None], w2[...],
                preferred_element_type=jnp.float32) + b2[...], 0.0)
    g3[...] = jnp.dot(x2 * on[:, None], w3[...],
                      preferred_element_type=jnp.float32)


def _tc_d_body(p3, g3, cd, b3, out):
    inn = _norm(cd)
    aggf = p3[0] + p3[1] + g3[...]
    out[...] = aggf * inn[:, None] + b3[...]


def _rows_spec(d=128):
    return pl.BlockSpec((BR, d), lambda i: (i, 0))


def _plane_spec():
    return pl.BlockSpec((2, BR, 128), lambda i: (0, i, 0))


def _cnt_spec():
    return pl.BlockSpec((32, BR), lambda i: (0, i))


def _full_spec(shape):
    return pl.BlockSpec(shape, lambda i: tuple(0 for _ in shape))


def _tc_call(body, in_specs, out_specs, out_shapes, args):
    return pl.pallas_call(
        body,
        grid=(NP_ // BR,),
        in_specs=in_specs,
        out_specs=out_specs,
        out_shape=out_shapes,
    )(*args)


def kernel(features, edge_index, W1, b1, W2, b2, W3, b3):
    src = edge_index[0].astype(jnp.int32)
    dst = edge_index[1].astype(jnp.int32)
    pad = jnp.full((EP - E,), DUMMY, jnp.int32)
    srcp = jnp.concatenate([src, pad])
    dstp = jnp.concatenate([dst, pad])
    feat_p = jnp.pad(features, ((0, NP_ - N), (0, 0)))
    b1r = b1.reshape(1, -1)
    b2r = b2.reshape(1, -1)
    b3r = b3.reshape(1, -1)

    cs, cd = _degrees(src, dst)

    h1 = _tc_call(
        _tc_a_body,
        [_rows_spec(), _cnt_spec()],
        _rows_spec(),
        jax.ShapeDtypeStruct((NP_, 128), jnp.float32),
        (feat_p, cs),
    )

    p1 = _agg_edges(h1, srcp, dstp)

    h2lo, h2hi = _tc_call(
        _tc_b_body,
        [_plane_spec(), _rows_spec(), _cnt_spec(), _cnt_spec(),
         _full_spec((D_IN, D_HID)), _full_spec((1, D_HID))],
        [_rows_spec(), _rows_spec()],
        [jax.ShapeDtypeStruct((NP_, 128), jnp.float32),
         jax.ShapeDtypeStruct((NP_, 128), jnp.float32)],
        (p1, h1, cs, cd, W1, b1r),
    )

    a2lo, a2hi = _agg_cols(h2lo, h2hi, srcp, dstp)

    g3 = _tc_call(
        _tc_c_body,
        [_rows_spec(), _rows_spec(), _rows_spec(), _rows_spec(),
         _cnt_spec(), _cnt_spec(), _full_spec((D_HID, D_HID)),
         _full_spec((1, D_HID)), _full_spec((D_HID, D_OUT))],
        _rows_spec(),
        jax.ShapeDtypeStruct((NP_, 128), jnp.float32),
        (a2lo, a2hi, h2lo, h2hi, cs, cd, W2, b2r, W3),
    )

    p3 = _agg_edges(g3, srcp, dstp)

    out = _tc_call(
        _tc_d_body,
        [_plane_spec(), _rows_spec(), _cnt_spec(), _full_spec((1, D_OUT))],
        _rows_spec(),
        jax.ShapeDtypeStruct((NP_, 128), jnp.float32),
        (p3, g3, cd, b3r),
    )
    return out[:N]


# speedup vs baseline: 1.1787x; 1.0317x over previous
import jax
import jax.numpy as jnp
from jax import lax
from jax.experimental import pallas as pl
from jax.experimental.pallas import tpu as pltpu
from jax.experimental.pallas import tpu_sc as plsc

N = 10000
E = 320000
D_IN = 128
D_HID = 256
D_OUT = 128

NP_ = 10240
EP = 323584
DUMMY = N
CHUNK = 128
RPS = NP_ // 16
BR = 512


_SC_PARAMS = pltpu.CompilerParams(needs_layout_passes=False)


def _deg_body(src_hbm, dst_hbm, out_s_hbm, out_d_hbm,
              ebuf_s, ebuf_d, cnt_s, cnt_d):
    c = lax.axis_index("c")
    s = lax.axis_index("s")
    wid = c * 16 + s
    zeros16 = jnp.zeros((16,), jnp.float32)
    ones16 = jnp.full((16,), 1.0, jnp.float32)

    def zrow(r, _):
        cnt_s[pl.ds(r * 16, 16)] = zeros16
        cnt_d[pl.ds(r * 16, 16)] = zeros16
        return 0
    lax.fori_loop(0, NP_ // 16, zrow, 0)

    for k in range(5):
        base = wid * 10000 + k * 2000
        pltpu.sync_copy(src_hbm.at[pl.ds(base, 2000)], ebuf_s)
        pltpu.sync_copy(dst_hbm.at[pl.ds(base, 2000)], ebuf_d)

        def cbody(i, _):
            sid = ebuf_s[pl.ds(i * 16, 16)]
            did = ebuf_d[pl.ds(i * 16, 16)]
            plsc.addupdate_scatter(cnt_s, [sid], ones16)
            plsc.addupdate_scatter(cnt_d, [did], ones16)
            return 0
        lax.fori_loop(0, 125, cbody, 0)

    pltpu.sync_copy(cnt_s, out_s_hbm.at[pl.ds(wid * NP_, NP_)])
    pltpu.sync_copy(cnt_d, out_d_hbm.at[pl.ds(wid * NP_, NP_)])


def _degrees(src, dst):
    mesh = plsc.VectorSubcoreMesh(core_axis_name="c", subcore_axis_name="s")
    f = pl.kernel(
        _deg_body,
        out_type=[jax.ShapeDtypeStruct((32 * NP_,), jnp.float32),
                  jax.ShapeDtypeStruct((32 * NP_,), jnp.float32)],
        mesh=mesh,
        compiler_params=_SC_PARAMS,
        scratch_types=[
            pltpu.VMEM((2000,), jnp.int32),
            pltpu.VMEM((2000,), jnp.int32),
            pltpu.VMEM((NP_,), jnp.float32),
            pltpu.VMEM((NP_,), jnp.float32),
        ],
    )
    cs, cd = f(src, dst)
    return cs.reshape(32, NP_), cd.reshape(32, NP_)


def _zero_acc(rows, acc, s):
    zeros16 = jnp.zeros((16,), jnp.float32)

    def zrow(r, _):
        for j in range(8):
            rows[r, pl.ds(16 * j, 16)] = zeros16
        return 0
    lax.fori_loop(0, CHUNK, zrow, 0)
    for k in range(RPS // CHUNK):
        pltpu.sync_copy(rows, acc.at[pl.ds(s * RPS + k * CHUNK, CHUNK)])


def _agg_chunks(h_hbm, srcp, dstp, sidx, didx, rows, acc, sem, tile_base,
                n_chunks, sidx2, didx2, rows2, sem2):
    def one(j, si, di, ro, se):
        base = tile_base + j * CHUNK
        pltpu.sync_copy(srcp.at[pl.ds(base, CHUNK)], si)
        pltpu.sync_copy(dstp.at[pl.ds(base, CHUNK)], di)
        return pltpu.async_copy(h_hbm.at[si], ro, se)

    def run_pairs(base_chunk, npairs):
        ga = one(base_chunk, sidx, didx, rows, sem)
        gb = one(base_chunk + 1, sidx2, didx2, rows2, sem2)
        for i in range(npairs):
            ga.wait()
            pltpu.sync_copy(rows, acc.at[didx], add=True)
            if i < npairs - 1:
                ga = one(base_chunk + 2 * i + 2, sidx, didx, rows, sem)
            gb.wait()
            pltpu.sync_copy(rows2, acc.at[didx2], add=True)
            if i < npairs - 1:
                gb = one(base_chunk + 2 * i + 3, sidx2, didx2, rows2, sem2)

    UNROLL = 4

    def body(q, _):
        run_pairs(2 * UNROLL * q, UNROLL)
        return 0
    lax.fori_loop(0, n_chunks // (2 * UNROLL), body, 0)
    rem = n_chunks % (2 * UNROLL)
    rem_base = n_chunks - rem
    if rem >= 2:
        run_pairs(rem_base, rem // 2)
    if rem % 2:
        one(n_chunks - 1, sidx, didx, rows, sem).wait()
        pltpu.sync_copy(rows, acc.at[didx], add=True)


def _agg_edges_body(h_hbm, srcp, dstp, out_hbm, sidx, didx, rows, acc, sem,
                    sidx2, didx2, rows2, sem2):
    c = lax.axis_index("c")
    s = lax.axis_index("s")
    _zero_acc(rows, acc, s)
    plsc.subcore_barrier()
    n_chunks = EP // 32 // CHUNK
    tile_base = (c * 16 + s) * (EP // 32)
    _agg_chunks(h_hbm, srcp, dstp, sidx, didx, rows, acc, sem, tile_base,
                n_chunks, sidx2, didx2, rows2, sem2)
    plsc.subcore_barrier()
    pltpu.sync_copy(acc.at[pl.ds(s * RPS, RPS)],
                    out_hbm.at[c, pl.ds(s * RPS, RPS)])


def _agg_cols_body(hlo_hbm, hhi_hbm, srcp, dstp, outlo_hbm, outhi_hbm,
                   sidx, didx, rows, acc, sem, sidx2, didx2, rows2, sem2):
    c = lax.axis_index("c")
    s = lax.axis_index("s")
    _zero_acc(rows, acc, s)
    plsc.subcore_barrier()
    n_chunks = EP // 16 // CHUNK
    tile_base = s * (EP // 16)

    @pl.when(c == 0)
    def _():
        _agg_chunks(hlo_hbm, srcp, dstp, sidx, didx, rows, acc, sem,
                    tile_base, n_chunks, sidx2, didx2, rows2, sem2)

    @pl.when(c == 1)
    def _():
        _agg_chunks(hhi_hbm, srcp, dstp, sidx, didx, rows, acc, sem,
                    tile_base, n_chunks, sidx2, didx2, rows2, sem2)

    plsc.subcore_barrier()

    @pl.when(c == 0)
    def _():
        pltpu.sync_copy(acc.at[pl.ds(s * RPS, RPS)],
                        outlo_hbm.at[pl.ds(s * RPS, RPS)])

    @pl.when(c == 1)
    def _():
        pltpu.sync_copy(acc.at[pl.ds(s * RPS, RPS)],
                        outhi_hbm.at[pl.ds(s * RPS, RPS)])


def _agg_scratch():
    return [
        pltpu.VMEM((CHUNK,), jnp.int32),
        pltpu.VMEM((CHUNK,), jnp.int32),
        pltpu.VMEM((CHUNK, 128), jnp.float32),
        pltpu.VMEM_SHARED((NP_, 128), jnp.float32),
        pltpu.SemaphoreType.DMA,
        pltpu.VMEM((CHUNK,), jnp.int32),
        pltpu.VMEM((CHUNK,), jnp.int32),
        pltpu.VMEM((CHUNK, 128), jnp.float32),
        pltpu.SemaphoreType.DMA,
    ]


def _agg_edges(h, srcp, dstp):
    mesh = plsc.VectorSubcoreMesh(core_axis_name="c", subcore_axis_name="s")
    f = pl.kernel(
        _agg_edges_body,
        out_type=jax.ShapeDtypeStruct((2, NP_, 128), jnp.float32),
        mesh=mesh,
        compiler_params=_SC_PARAMS,
        scratch_types=_agg_scratch(),
    )
    return f(h, srcp, dstp)


def _agg_cols(hlo, hhi, srcp, dstp):
    mesh = plsc.VectorSubcoreMesh(core_axis_name="c", subcore_axis_name="s")
    f = pl.kernel(
        _agg_cols_body,
        out_type=[jax.ShapeDtypeStruct((NP_, 128), jnp.float32),
                  jax.ShapeDtypeStruct((NP_, 128), jnp.float32)],
        mesh=mesh,
        compiler_params=_SC_PARAMS,
        scratch_types=_agg_scratch(),
    )
    return f(hlo, hhi, srcp, dstp)


def _norm(cnt_blk):
    return lax.rsqrt(jnp.sum(cnt_blk[...], axis=0) + 1.0)


def _tc_a_body(feat, cs, h1):
    on = _norm(cs)
    h1[...] = feat[...] * on[:, None]


def _tc_b_body(p1, h1, cs, cd, w1, b1, h2lo, h2hi):
    on, inn = _norm(cs), _norm(cd)
    aggf = p1[0] + p1[1] + h1[...]
    x1 = jnp.dot(aggf * inn[:, None], w1[...],
                 preferred_element_type=jnp.float32) + b1[...]
    h2 = jnp.maximum(x1, 0.0) * on[:, None]
    h2lo[...] = h2[:, :128]
    h2hi[...] = h2[:, 128:]


def _tc_c_body(a2lo, a2hi, h2lo, h2hi, cs, cd, w2, b2, w3, g3):
    on, inn = _norm(cs), _norm(cd)
    aggf = jnp.concatenate([a2lo[...] + h2lo[...], a2hi[...] + h2hi[...]],
                           axis=1)
    x2 = jnp.maximum(
        jnp.dot(aggf * inn[:, None], w2[...],
                preferred_element_type=jnp.float32) + b2[...], 0.0)
    g3[...] = jnp.dot(x2 * on[:, None], w3[...],
                      preferred_element_type=jnp.float32)


def _tc_d_body(p3, g3, cd, b3, out):
    inn = _norm(cd)
    aggf = p3[0] + p3[1] + g3[...]
    out[...] = aggf * inn[:, None] + b3[...]


def _rows_spec(d=128):
    return pl.BlockSpec((BR, d), lambda i: (i, 0))


def _plane_spec():
    return pl.BlockSpec((2, BR, 128), lambda i: (0, i, 0))


def _cnt_spec():
    return pl.BlockSpec((32, BR), lambda i: (0, i))


def _full_spec(shape):
    return pl.BlockSpec(shape, lambda i: tuple(0 for _ in shape))


def _tc_call(body, in_specs, out_specs, out_shapes, args):
    return pl.pallas_call(
        body,
        grid=(NP_ // BR,),
        in_specs=in_specs,
        out_specs=out_specs,
        out_shape=out_shapes,
    )(*args)


def kernel(features, edge_index, W1, b1, W2, b2, W3, b3):
    src = edge_index[0].astype(jnp.int32)
    dst = edge_index[1].astype(jnp.int32)
    pad = jnp.full((EP - E,), DUMMY, jnp.int32)
    srcp = jnp.concatenate([src, pad])
    dstp = jnp.concatenate([dst, pad])
    feat_p = jnp.pad(features, ((0, NP_ - N), (0, 0)))
    b1r = b1.reshape(1, -1)
    b2r = b2.reshape(1, -1)
    b3r = b3.reshape(1, -1)

    cs, cd = _degrees(src, dst)

    h1 = _tc_call(
        _tc_a_body,
        [_rows_spec(), _cnt_spec()],
        _rows_spec(),
        jax.ShapeDtypeStruct((NP_, 128), jnp.float32),
        (feat_p, cs),
    )

    p1 = _agg_edges(h1, srcp, dstp)

    h2lo, h2hi = _tc_call(
        _tc_b_body,
        [_plane_spec(), _rows_spec(), _cnt_spec(), _cnt_spec(),
         _full_spec((D_IN, D_HID)), _full_spec((1, D_HID))],
        [_rows_spec(), _rows_spec()],
        [jax.ShapeDtypeStruct((NP_, 128), jnp.float32),
         jax.ShapeDtypeStruct((NP_, 128), jnp.float32)],
        (p1, h1, cs, cd, W1, b1r),
    )

    a2lo, a2hi = _agg_cols(h2lo, h2hi, srcp, dstp)

    g3 = _tc_call(
        _tc_c_body,
        [_rows_spec(), _rows_spec(), _rows_spec(), _rows_spec(),
         _cnt_spec(), _cnt_spec(), _full_spec((D_HID, D_HID)),
         _full_spec((1, D_HID)), _full_spec((D_HID, D_OUT))],
        _rows_spec(),
        jax.ShapeDtypeStruct((NP_, 128), jnp.float32),
        (a2lo, a2hi, h2lo, h2hi, cs, cd, W2, b2r, W3),
    )

    p3 = _agg_edges(g3, srcp, dstp)

    out = _tc_call(
        _tc_d_body,
        [_plane_spec(), _rows_spec(), _cnt_spec(), _full_spec((1, D_OUT))],
        _rows_spec(),
        jax.ShapeDtypeStruct((NP_, 128), jnp.float32),
        (p3, g3, cd, b3r),
    )
    return out[:N]


# speedup vs baseline: 1.1956x; 1.0143x over previous
import jax
import jax.numpy as jnp
from jax import lax
from jax.experimental import pallas as pl
from jax.experimental.pallas import tpu as pltpu
from jax.experimental.pallas import tpu_sc as plsc

N = 10000
E = 320000
D_IN = 128
D_HID = 256
D_OUT = 128

NP_ = 10240
EP = 323584
DUMMY = N
CHUNK = 128
RPS = NP_ // 16
BR = 512


_SC_PARAMS = pltpu.CompilerParams(needs_layout_passes=False)


def _deg_body(src_hbm, dst_hbm, out_s_hbm, out_d_hbm,
              ebuf_s, ebuf_d, cnt_s, cnt_d):
    c = lax.axis_index("c")
    s = lax.axis_index("s")
    wid = c * 16 + s
    zeros16 = jnp.zeros((16,), jnp.float32)
    ones16 = jnp.full((16,), 1.0, jnp.float32)

    def zrow(r, _):
        cnt_s[pl.ds(r * 16, 16)] = zeros16
        cnt_d[pl.ds(r * 16, 16)] = zeros16
        return 0
    lax.fori_loop(0, NP_ // 16, zrow, 0)

    for k in range(5):
        base = wid * 10000 + k * 2000
        pltpu.sync_copy(src_hbm.at[pl.ds(base, 2000)], ebuf_s)
        pltpu.sync_copy(dst_hbm.at[pl.ds(base, 2000)], ebuf_d)

        def cbody(i, _):
            sid = ebuf_s[pl.ds(i * 16, 16)]
            did = ebuf_d[pl.ds(i * 16, 16)]
            plsc.addupdate_scatter(cnt_s, [sid], ones16)
            plsc.addupdate_scatter(cnt_d, [did], ones16)
            return 0
        lax.fori_loop(0, 125, cbody, 0)

    pltpu.sync_copy(cnt_s, out_s_hbm.at[pl.ds(wid * NP_, NP_)])
    pltpu.sync_copy(cnt_d, out_d_hbm.at[pl.ds(wid * NP_, NP_)])


def _degrees(src, dst):
    mesh = plsc.VectorSubcoreMesh(core_axis_name="c", subcore_axis_name="s")
    f = pl.kernel(
        _deg_body,
        out_type=[jax.ShapeDtypeStruct((32 * NP_,), jnp.float32),
                  jax.ShapeDtypeStruct((32 * NP_,), jnp.float32)],
        mesh=mesh,
        compiler_params=_SC_PARAMS,
        scratch_types=[
            pltpu.VMEM((2000,), jnp.int32),
            pltpu.VMEM((2000,), jnp.int32),
            pltpu.VMEM((NP_,), jnp.float32),
            pltpu.VMEM((NP_,), jnp.float32),
        ],
    )
    cs, cd = f(src, dst)
    return cs.reshape(32, NP_), cd.reshape(32, NP_)


def _zero_acc(rows, acc, s):
    zeros16 = jnp.zeros((16,), jnp.float32)

    def zrow(r, _):
        for j in range(8):
            rows[r, pl.ds(16 * j, 16)] = zeros16
        return 0
    lax.fori_loop(0, CHUNK, zrow, 0)
    for k in range(RPS // CHUNK):
        pltpu.sync_copy(rows, acc.at[pl.ds(s * RPS + k * CHUNK, CHUNK)])


def _agg_chunks(h_hbm, srcp, dstp, sidx, didx, rows, acc, sem, tile_base,
                n_chunks, sidx2, didx2, rows2, sem2):
    def one(j, si, di, ro, se):
        base = tile_base + j * CHUNK
        pltpu.sync_copy(srcp.at[pl.ds(base, CHUNK)], si)
        pltpu.sync_copy(dstp.at[pl.ds(base, CHUNK)], di)
        return pltpu.async_copy(h_hbm.at[si], ro, se)

    def run_pairs(base_chunk, npairs):
        ga = one(base_chunk, sidx, didx, rows, sem)
        gb = one(base_chunk + 1, sidx2, didx2, rows2, sem2)
        for i in range(npairs):
            ga.wait()
            pltpu.sync_copy(rows, acc.at[didx], add=True)
            if i < npairs - 1:
                ga = one(base_chunk + 2 * i + 2, sidx, didx, rows, sem)
            gb.wait()
            pltpu.sync_copy(rows2, acc.at[didx2], add=True)
            if i < npairs - 1:
                gb = one(base_chunk + 2 * i + 3, sidx2, didx2, rows2, sem2)

    UNROLL = 8

    def body(q, _):
        run_pairs(2 * UNROLL * q, UNROLL)
        return 0
    lax.fori_loop(0, n_chunks // (2 * UNROLL), body, 0)
    rem = n_chunks % (2 * UNROLL)
    rem_base = n_chunks - rem
    if rem >= 2:
        run_pairs(rem_base, rem // 2)
    if rem % 2:
        one(n_chunks - 1, sidx, didx, rows, sem).wait()
        pltpu.sync_copy(rows, acc.at[didx], add=True)


def _agg_edges_body(h_hbm, srcp, dstp, out_hbm, sidx, didx, rows, acc, sem,
                    sidx2, didx2, rows2, sem2):
    c = lax.axis_index("c")
    s = lax.axis_index("s")
    _zero_acc(rows, acc, s)
    plsc.subcore_barrier()
    n_chunks = EP // 32 // CHUNK
    tile_base = (c * 16 + s) * (EP // 32)
    _agg_chunks(h_hbm, srcp, dstp, sidx, didx, rows, acc, sem, tile_base,
                n_chunks, sidx2, didx2, rows2, sem2)
    plsc.subcore_barrier()
    pltpu.sync_copy(acc.at[pl.ds(s * RPS, RPS)],
                    out_hbm.at[c, pl.ds(s * RPS, RPS)])


def _agg_cols_body(hlo_hbm, hhi_hbm, srcp, dstp, outlo_hbm, outhi_hbm,
                   sidx, didx, rows, acc, sem, sidx2, didx2, rows2, sem2):
    c = lax.axis_index("c")
    s = lax.axis_index("s")
    _zero_acc(rows, acc, s)
    plsc.subcore_barrier()
    n_chunks = EP // 16 // CHUNK
    tile_base = s * (EP // 16)

    @pl.when(c == 0)
    def _():
        _agg_chunks(hlo_hbm, srcp, dstp, sidx, didx, rows, acc, sem,
                    tile_base, n_chunks, sidx2, didx2, rows2, sem2)

    @pl.when(c == 1)
    def _():
        _agg_chunks(hhi_hbm, srcp, dstp, sidx, didx, rows, acc, sem,
                    tile_base, n_chunks, sidx2, didx2, rows2, sem2)

    plsc.subcore_barrier()

    @pl.when(c == 0)
    def _():
        pltpu.sync_copy(acc.at[pl.ds(s * RPS, RPS)],
                        outlo_hbm.at[pl.ds(s * RPS, RPS)])

    @pl.when(c == 1)
    def _():
        pltpu.sync_copy(acc.at[pl.ds(s * RPS, RPS)],
                        outhi_hbm.at[pl.ds(s * RPS, RPS)])


def _agg_scratch():
    return [
        pltpu.VMEM((CHUNK,), jnp.int32),
        pltpu.VMEM((CHUNK,), jnp.int32),
        pltpu.VMEM((CHUNK, 128), jnp.float32),
        pltpu.VMEM_SHARED((NP_, 128), jnp.float32),
        pltpu.SemaphoreType.DMA,
        pltpu.VMEM((CHUNK,), jnp.int32),
        pltpu.VMEM((CHUNK,), jnp.int32),
        pltpu.VMEM((CHUNK, 128), jnp.float32),
        pltpu.SemaphoreType.DMA,
    ]


def _agg_edges(h, srcp, dstp):
    mesh = plsc.VectorSubcoreMesh(core_axis_name="c", subcore_axis_name="s")
    f = pl.kernel(
        _agg_edges_body,
        out_type=jax.ShapeDtypeStruct((2, NP_, 128), jnp.float32),
        mesh=mesh,
        compiler_params=_SC_PARAMS,
        scratch_types=_agg_scratch(),
    )
    return f(h, srcp, dstp)


def _agg_cols(hlo, hhi, srcp, dstp):
    mesh = plsc.VectorSubcoreMesh(core_axis_name="c", subcore_axis_name="s")
    f = pl.kernel(
        _agg_cols_body,
        out_type=[jax.ShapeDtypeStruct((NP_, 128), jnp.float32),
                  jax.ShapeDtypeStruct((NP_, 128), jnp.float32)],
        mesh=mesh,
        compiler_params=_SC_PARAMS,
        scratch_types=_agg_scratch(),
    )
    return f(hlo, hhi, srcp, dstp)


def _norm(cnt_blk):
    return lax.rsqrt(jnp.sum(cnt_blk[...], axis=0) + 1.0)


def _tc_a_body(feat, cs, h1):
    on = _norm(cs)
    h1[...] = feat[...] * on[:, None]


def _tc_b_body(p1, h1, cs, cd, w1, b1, h2lo, h2hi):
    on, inn = _norm(cs), _norm(cd)
    aggf = p1[0] + p1[1] + h1[...]
    x1 = jnp.dot(aggf * inn[:, None], w1[...],
                 preferred_element_type=jnp.float32) + b1[...]
    h2 = jnp.maximum(x1, 0.0) * on[:, None]
    h2lo[...] = h2[:, :128]
    h2hi[...] = h2[:, 128:]


def _tc_c_body(a2lo, a2hi, h2lo, h2hi, cs, cd, w2, b2, w3, g3):
    on, inn = _norm(cs), _norm(cd)
    aggf = jnp.concatenate([a2lo[...] + h2lo[...], a2hi[...] + h2hi[...]],
                           axis=1)
    x2 = jnp.maximum(
        jnp.dot(aggf * inn[:, None], w2[...],
                preferred_element_type=jnp.float32) + b2[...], 0.0)
    g3[...] = jnp.dot(x2 * on[:, None], w3[...],
                      preferred_element_type=jnp.float32)


def _tc_d_body(p3, g3, cd, b3, out):
    inn = _norm(cd)
    aggf = p3[0] + p3[1] + g3[...]
    out[...] = aggf * inn[:, None] + b3[...]


def _rows_spec(d=128):
    return pl.BlockSpec((BR, d), lambda i: (i, 0))


def _plane_spec():
    return pl.BlockSpec((2, BR, 128), lambda i: (0, i, 0))


def _cnt_spec():
    return pl.BlockSpec((32, BR), lambda i: (0, i))


def _full_spec(shape):
    return pl.BlockSpec(shape, lambda i: tuple(0 for _ in shape))


def _tc_call(body, in_specs, out_specs, out_shapes, args):
    return pl.pallas_call(
        body,
        grid=(NP_ // BR,),
        in_specs=in_specs,
        out_specs=out_specs,
        out_shape=out_shapes,
    )(*args)


def kernel(features, edge_index, W1, b1, W2, b2, W3, b3):
    src = edge_index[0].astype(jnp.int32)
    dst = edge_index[1].astype(jnp.int32)
    pad = jnp.full((EP - E,), DUMMY, jnp.int32)
    srcp = jnp.concatenate([src, pad])
    dstp = jnp.concatenate([dst, pad])
    feat_p = jnp.pad(features, ((0, NP_ - N), (0, 0)))
    b1r = b1.reshape(1, -1)
    b2r = b2.reshape(1, -1)
    b3r = b3.reshape(1, -1)

    cs, cd = _degrees(src, dst)

    h1 = _tc_call(
        _tc_a_body,
        [_rows_spec(), _cnt_spec()],
        _rows_spec(),
        jax.ShapeDtypeStruct((NP_, 128), jnp.float32),
        (feat_p, cs),
    )

    p1 = _agg_edges(h1, srcp, dstp)

    h2lo, h2hi = _tc_call(
        _tc_b_body,
        [_plane_spec(), _rows_spec(), _cnt_spec(), _cnt_spec(),
         _full_spec((D_IN, D_HID)), _full_spec((1, D_HID))],
        [_rows_spec(), _rows_spec()],
        [jax.ShapeDtypeStruct((NP_, 128), jnp.float32),
         jax.ShapeDtypeStruct((NP_, 128), jnp.float32)],
        (p1, h1, cs, cd, W1, b1r),
    )

    a2lo, a2hi = _agg_cols(h2lo, h2hi, srcp, dstp)

    g3 = _tc_call(
        _tc_c_body,
        [_rows_spec(), _rows_spec(), _rows_spec(), _rows_spec(),
         _cnt_spec(), _cnt_spec(), _full_spec((D_HID, D_HID)),
         _full_spec((1, D_HID)), _full_spec((D_HID, D_OUT))],
        _rows_spec(),
        jax.ShapeDtypeStruct((NP_, 128), jnp.float32),
        (a2lo, a2hi, h2lo, h2hi, cs, cd, W2, b2r, W3),
    )

    p3 = _agg_edges(g3, srcp, dstp)

    out = _tc_call(
        _tc_d_body,
        [_plane_spec(), _rows_spec(), _cnt_spec(), _full_spec((1, D_OUT))],
        _rows_spec(),
        jax.ShapeDtypeStruct((NP_, 128), jnp.float32),
        (p3, g3, cd, b3r),
    )
    return out[:N]
